# baseline (speedup 1.0000x reference)
import functools

import jax
import jax.numpy as jnp
from jax import lax
from jax.experimental import pallas as pl
from jax.experimental.pallas import tpu as pltpu
from jax.experimental.pallas import tpu_sc as plsc

B = 64
C = 512
H = 14
W = 14
K = 3
N_FIND = 64
N_ROOT = 16
N_ANS = 2000
V = 5000
L = 20
D = 256
HID = 512
HW = H * W

BB = 8
TN = 2000
NT = 1


def _find_body(feat_ref, ohf_ref, wf_ref, bf_ref, maps_ref, att_ref):
    wf = wf_ref[...]
    bf = bf_ref[...]
    for s in range(BB):
        oh = ohf_ref[s]
        wk = jnp.dot(oh, wf, preferred_element_type=jnp.float32)
        bk = jnp.dot(oh, bf, preferred_element_type=jnp.float32)
        feat = feat_ref[s]
        a = jnp.dot(wk, feat, preferred_element_type=jnp.float32) + bk
        a = jnp.maximum(a, 0.0)
        m = a[0:1] * a[1:2] * a[2:3]
        maps_ref[s, :] = m[0]
        att_ref[s, :] = jnp.sum(feat * m, axis=1)


def _find_call(feat, ohf, w_find, b_find2d):
    return pl.pallas_call(
        _find_body,
        grid=(B // BB,),
        in_specs=[
            pl.BlockSpec((BB, C, HW), lambda i: (i, 0, 0)),
            pl.BlockSpec((BB, K, N_FIND), lambda i: (i, 0, 0)),
            pl.BlockSpec((N_FIND, C), lambda i: (0, 0)),
            pl.BlockSpec((N_FIND, 1), lambda i: (0, 0)),
        ],
        out_specs=[
            pl.BlockSpec((BB, HW), lambda i: (i, 0)),
            pl.BlockSpec((BB, C), lambda i: (i, 0)),
        ],
        out_shape=[
            jax.ShapeDtypeStruct((B, HW), jnp.float32),
            jax.ShapeDtypeStruct((B, C), jnp.float32),
        ],
        compiler_params=pltpu.CompilerParams(
            dimension_semantics=("arbitrary",)),
    )(feat, ohf, w_find, b_find2d)


def _root_body(maps_ref, att_ref, ri_ref, ys_ref, w1_ref, b1_ref,
               w2_ref, b2_ref, wd_ref, bd_ref, out_ref):
    e = pl.program_id(0)
    sel = (ri_ref[...] == e).astype(jnp.float32)
    ys = ys_ref[...]
    sy = sel * ys
    sn = sel * (1.0 - ys)
    maps = maps_ref[...] * sel
    h1 = jnp.dot(maps, w1_ref[0], preferred_element_type=jnp.float32)
    h1 = jnp.maximum(h1 + b1_ref[0], 0.0) * sy
    h1 = h1.astype(jnp.bfloat16)
    att = (att_ref[...] * sn).astype(jnp.bfloat16)

    @pl.when(e == 0)
    def _():
        out_ref[...] = jnp.zeros_like(out_ref)

    out_ref[...] += (
        jnp.dot(h1, w2_ref[...].astype(jnp.bfloat16),
                preferred_element_type=jnp.float32)
        + jnp.dot(att, wd_ref[...].astype(jnp.bfloat16),
                  preferred_element_type=jnp.float32)
        + sy * b2_ref[0] + sn * bd_ref[0])


def _root_call(maps, att, ri, ys, w1, b1, w2, b2p, wd, bdp):
    return pl.pallas_call(
        _root_body,
        grid=(N_ROOT,),
        in_specs=[
            pl.BlockSpec((B, HW), lambda e: (0, 0)),
            pl.BlockSpec((B, C), lambda e: (0, 0)),
            pl.BlockSpec((B, 1), lambda e: (0, 0)),
            pl.BlockSpec((B, 1), lambda e: (0, 0)),
            pl.BlockSpec((1, HW, HID), lambda e: (e, 0, 0)),
            pl.BlockSpec((1, 1, HID), lambda e: (e, 0, 0)),
            pl.BlockSpec((HID, N_ANS), lambda e: (e, 0)),
            pl.BlockSpec((1, 1, N_ANS), lambda e: (e, 0, 0)),
            pl.BlockSpec((C, N_ANS), lambda e: (e, 0)),
            pl.BlockSpec((1, 1, N_ANS), lambda e: (e, 0, 0)),
        ],
        out_specs=pl.BlockSpec((B, N_ANS), lambda e: (0, 0)),
        out_shape=jax.ShapeDtypeStruct((B, N_ANS), jnp.float32),
        compiler_params=pltpu.CompilerParams(
            dimension_semantics=("arbitrary",)),
    )(maps, att, ri, ys, w1, b1, w2, b2p, wd, bdp)


def _lstm_body(embt_ref, wi_ref, wh_ref, bl_ref, idx_ref, wout_ref, bout_ref,
               rl_ref, out_ref, xw_ref):
    xw_ref[...] = jnp.dot(embt_ref[...], wi_ref[...],
                          preferred_element_type=jnp.float32)
    wh = wh_ref[...]
    bl = bl_ref[...]
    idx = idx_ref[...]

    def step(t, carry):
        h, c, hf = carry
        z = xw_ref[pl.ds(t * B, B), :] + jnp.dot(
            h, wh, preferred_element_type=jnp.float32) + bl
        i = jax.nn.sigmoid(z[:, :D])
        f = jax.nn.sigmoid(z[:, D:2 * D])
        g = jnp.tanh(z[:, 2 * D:3 * D])
        o = jax.nn.sigmoid(z[:, 3 * D:])
        c = f * c + i * g
        h = o * jnp.tanh(c)
        hf = hf + (idx == t).astype(jnp.float32) * h
        return (h, c, hf)

    h0 = jnp.zeros((B, D), jnp.float32)
    _, _, hf = lax.fori_loop(0, L, step, (h0, h0, h0))
    el = jnp.dot(hf, wout_ref[...], preferred_element_type=jnp.float32)
    el = el + bout_ref[...]
    pe = jnp.exp(el - jnp.max(el, axis=1, keepdims=True))
    pe = pe / jnp.sum(pe, axis=1, keepdims=True)
    rl = rl_ref[...]
    pr = jnp.exp(rl - jnp.max(rl, axis=1, keepdims=True))
    pr = pr / jnp.sum(pr, axis=1, keepdims=True)
    out_ref[...] = jnp.sqrt(pe * pr)


def _lstm_call(embt, wi, wh, bl, idx, wout, bout, rlogits):
    args = (embt, wi, wh, bl, idx, wout, bout, rlogits)
    return pl.pallas_call(
        _lstm_body,
        in_specs=[pl.BlockSpec(x.shape, functools.partial(lambda n: (0,) * n,
                                                          x.ndim))
                  for x in args],
        out_specs=pl.BlockSpec((B, N_ANS), lambda: (0, 0)),
        out_shape=jax.ShapeDtypeStruct((B, N_ANS), jnp.float32),
        scratch_shapes=[pltpu.VMEM((L * B, 4 * D), jnp.float32)],
    )(*args)


def _emb_gather(embed, qidx_flat):
    info = plsc.get_sparse_core_info()
    nw = info.num_cores * info.num_subcores
    bpw = (B * L) // nw
    nc = info.num_cores
    mesh = plsc.VectorSubcoreMesh(core_axis_name="c", subcore_axis_name="s")

    @functools.partial(
        pl.kernel, mesh=mesh,
        out_type=jax.ShapeDtypeStruct((B * L, D), jnp.float32),
        scratch_types=[
            pltpu.VMEM((bpw,), jnp.int32),
            pltpu.VMEM((bpw, D), jnp.float32),
            pltpu.SemaphoreType.DMA,
        ],
    )
    def k(table_hbm, idx_hbm, out_hbm, idx_v, rows_v, sem):
        wid = lax.axis_index("s") * nc + lax.axis_index("c")
        base = wid * bpw
        pltpu.sync_copy(idx_hbm.at[pl.ds(base, bpw)], idx_v)
        pltpu.async_copy(table_hbm.at[idx_v], rows_v, sem).wait()
        pltpu.sync_copy(rows_v, out_hbm.at[pl.ds(base, bpw)])

    return k(embed, qidx_flat)


def kernel(features, question, length, yesno, root_inst, find_inst,
           W_find, b_find, W_meas1, b_meas1, W_meas2, b_meas2,
           W_desc, b_desc, embed, Wi, Wh, b_lstm, W_out, b_out):
    feat = features.reshape(B, C, HW)
    ohf = (find_inst[:, :, None]
           == jnp.arange(N_FIND, dtype=find_inst.dtype)).astype(jnp.float32)
    maps, att = _find_call(feat, ohf, W_find, b_find.reshape(N_FIND, 1))

    ri = root_inst.astype(jnp.int32).reshape(B, 1)
    ys = yesno.astype(jnp.float32).reshape(B, 1)
    b2p = b_meas2.reshape(N_ROOT, 1, N_ANS)
    bdp = b_desc.reshape(N_ROOT, 1, N_ANS)
    rlogits = _root_call(maps, att, ri, ys,
                         W_meas1,
                         b_meas1.reshape(N_ROOT, 1, HID),
                         W_meas2.reshape(N_ROOT * HID, N_ANS), b2p,
                         W_desc.reshape(N_ROOT * C, N_ANS), bdp)

    emb = _emb_gather(embed, question.reshape(-1).astype(jnp.int32))
    embt = emb.reshape(B, L, D).transpose(1, 0, 2).reshape(L * B, D)
    idx = (jnp.clip(length, 1, L) - 1).astype(jnp.int32).reshape(B, 1)
    return _lstm_call(embt, Wi, Wh, b_lstm.reshape(1, 4 * D), idx,
                      W_out, b_out.reshape(1, N_ANS), rlogits)


# speedup vs baseline: 1.3807x; 1.3807x over previous
import functools

import jax
import jax.numpy as jnp
from jax import lax
from jax.experimental import pallas as pl
from jax.experimental.pallas import tpu as pltpu
from jax.experimental.pallas import tpu_sc as plsc

B = 64
C = 512
H = 14
W = 14
K = 3
N_FIND = 64
N_ROOT = 16
N_ANS = 2000
V = 5000
L = 20
D = 256
HID = 512
HW = H * W

BB = 8
TN = 2000
NT = 1


def _find_body(feat_ref, ohf_ref, wf_ref, bf_ref, maps_ref, att_ref):
    wf = wf_ref[...]
    bf = bf_ref[...]
    for s in range(BB):
        oh = ohf_ref[s]
        wk = jnp.dot(oh, wf, preferred_element_type=jnp.float32)
        bk = jnp.dot(oh, bf, preferred_element_type=jnp.float32)
        feat = feat_ref[s]
        a = jnp.dot(wk, feat, preferred_element_type=jnp.float32) + bk
        a = jnp.maximum(a, 0.0)
        m = a[0:1] * a[1:2] * a[2:3]
        maps_ref[s, :] = m[0]
        att_ref[s, :] = jnp.sum(feat * m, axis=1)


def _find_call(feat, ohf, w_find, b_find2d):
    return pl.pallas_call(
        _find_body,
        grid=(B // BB,),
        in_specs=[
            pl.BlockSpec((BB, C, HW), lambda i: (i, 0, 0)),
            pl.BlockSpec((BB, K, N_FIND), lambda i: (i, 0, 0)),
            pl.BlockSpec((N_FIND, C), lambda i: (0, 0)),
            pl.BlockSpec((N_FIND, 1), lambda i: (0, 0)),
        ],
        out_specs=[
            pl.BlockSpec((BB, HW), lambda i: (i, 0)),
            pl.BlockSpec((BB, C), lambda i: (i, 0)),
        ],
        out_shape=[
            jax.ShapeDtypeStruct((B, HW), jnp.float32),
            jax.ShapeDtypeStruct((B, C), jnp.float32),
        ],
        compiler_params=pltpu.CompilerParams(
            dimension_semantics=("arbitrary",)),
    )(feat, ohf, w_find, b_find2d)


def _root_body(maps_ref, att_ref, ri_ref, ys_ref, w1_ref, b1_ref,
               w2t_ref, wdt_ref, outt_ref):
    e = pl.program_id(0)
    sel = (ri_ref[...] == e).astype(jnp.float32)
    ys = ys_ref[...]
    sy = sel * ys
    sn = sel * (1.0 - ys)
    maps = maps_ref[...] * sel
    h1 = jnp.dot(maps, w1_ref[0], preferred_element_type=jnp.float32)
    h1 = jnp.maximum(h1 + b1_ref[0], 0.0) * sy
    h1t = h1.T.astype(jnp.bfloat16)
    attt = (att_ref[...] * sn).T.astype(jnp.bfloat16)

    @pl.when(e == 0)
    def _():
        outt_ref[...] = jnp.zeros_like(outt_ref)

    outt_ref[...] += (
        jnp.dot(w2t_ref[...].astype(jnp.bfloat16), h1t,
                preferred_element_type=jnp.float32)
        + jnp.dot(wdt_ref[...].astype(jnp.bfloat16), attt,
                  preferred_element_type=jnp.float32))


def _root_call(maps, att, ri, ys, w1, b1, w2t, wdt):
    return pl.pallas_call(
        _root_body,
        grid=(N_ROOT,),
        in_specs=[
            pl.BlockSpec((B, HW), lambda e: (0, 0)),
            pl.BlockSpec((B, C), lambda e: (0, 0)),
            pl.BlockSpec((B, 1), lambda e: (0, 0)),
            pl.BlockSpec((B, 1), lambda e: (0, 0)),
            pl.BlockSpec((1, HW, HID), lambda e: (e, 0, 0)),
            pl.BlockSpec((1, 1, HID), lambda e: (e, 0, 0)),
            pl.BlockSpec((N_ANS, HID), lambda e: (e, 0)),
            pl.BlockSpec((N_ANS, C), lambda e: (e, 0)),
        ],
        out_specs=pl.BlockSpec((N_ANS, B), lambda e: (0, 0)),
        out_shape=jax.ShapeDtypeStruct((N_ANS, B), jnp.float32),
        compiler_params=pltpu.CompilerParams(
            dimension_semantics=("arbitrary",)),
    )(maps, att, ri, ys, w1, b1, w2t, wdt)


def _lstm_body(embt_ref, wi_ref, wh_ref, bl_ref, idx_ref, wout_ref, bout_ref,
               rlt_ref, ri_ref, ys_ref, b2_ref, bd_ref, out_ref, xw_ref):
    xw_ref[...] = jnp.dot(embt_ref[...], wi_ref[...],
                          preferred_element_type=jnp.float32)
    wh = wh_ref[...]
    bl = bl_ref[...]
    idx = idx_ref[...]

    def step(t, carry):
        h, c, hf = carry
        z = xw_ref[pl.ds(t * B, B), :] + jnp.dot(
            h, wh, preferred_element_type=jnp.float32) + bl
        i = jax.nn.sigmoid(z[:, :D])
        f = jax.nn.sigmoid(z[:, D:2 * D])
        g = jnp.tanh(z[:, 2 * D:3 * D])
        o = jax.nn.sigmoid(z[:, 3 * D:])
        c = f * c + i * g
        h = o * jnp.tanh(c)
        hf = hf + (idx == t).astype(jnp.float32) * h
        return (h, c, hf)

    h0 = jnp.zeros((B, D), jnp.float32)
    _, _, hf = lax.fori_loop(0, L, step, (h0, h0, h0))
    el = jnp.dot(hf, wout_ref[...], preferred_element_type=jnp.float32)
    el = el + bout_ref[...]
    pe = jnp.exp(el - jnp.max(el, axis=1, keepdims=True))
    pe = pe / jnp.sum(pe, axis=1, keepdims=True)
    oh = (ri_ref[...] == lax.broadcasted_iota(jnp.int32, (B, N_ROOT), 1)
          ).astype(jnp.float32)
    ys = ys_ref[...]
    bias = (jnp.dot(oh * ys, b2_ref[...], preferred_element_type=jnp.float32)
            + jnp.dot(oh * (1.0 - ys), bd_ref[...],
                      preferred_element_type=jnp.float32))
    rl = rlt_ref[...].T + bias
    pr = jnp.exp(rl - jnp.max(rl, axis=1, keepdims=True))
    pr = pr / jnp.sum(pr, axis=1, keepdims=True)
    out_ref[...] = jnp.sqrt(pe * pr)


def _lstm_call(embt, wi, wh, bl, idx, wout, bout, rlt, ri, ys, b2, bd):
    args = (embt, wi, wh, bl, idx, wout, bout, rlt, ri, ys, b2, bd)
    return pl.pallas_call(
        _lstm_body,
        in_specs=[pl.BlockSpec(x.shape, functools.partial(lambda n: (0,) * n,
                                                          x.ndim))
                  for x in args],
        out_specs=pl.BlockSpec((B, N_ANS), lambda: (0, 0)),
        out_shape=jax.ShapeDtypeStruct((B, N_ANS), jnp.float32),
        scratch_shapes=[pltpu.VMEM((L * B, 4 * D), jnp.float32)],
    )(*args)


def _emb_gather(embed, qidx_flat):
    info = plsc.get_sparse_core_info()
    nw = info.num_cores * info.num_subcores
    bpw = (B * L) // nw
    nc = info.num_cores
    mesh = plsc.VectorSubcoreMesh(core_axis_name="c", subcore_axis_name="s")

    @functools.partial(
        pl.kernel, mesh=mesh,
        out_type=jax.ShapeDtypeStruct((B * L, D), jnp.float32),
        scratch_types=[
            pltpu.VMEM((bpw,), jnp.int32),
            pltpu.VMEM((bpw, D), jnp.float32),
            pltpu.SemaphoreType.DMA,
        ],
    )
    def k(table_hbm, idx_hbm, out_hbm, idx_v, rows_v, sem):
        wid = lax.axis_index("s") * nc + lax.axis_index("c")
        base = wid * bpw
        pltpu.sync_copy(idx_hbm.at[pl.ds(base, bpw)], idx_v)
        pltpu.async_copy(table_hbm.at[idx_v], rows_v, sem).wait()
        pltpu.sync_copy(rows_v, out_hbm.at[pl.ds(base, bpw)])

    return k(embed, qidx_flat)


def kernel(features, question, length, yesno, root_inst, find_inst,
           W_find, b_find, W_meas1, b_meas1, W_meas2, b_meas2,
           W_desc, b_desc, embed, Wi, Wh, b_lstm, W_out, b_out):
    feat = features.reshape(B, C, HW)
    ohf = (find_inst[:, :, None]
           == jnp.arange(N_FIND, dtype=find_inst.dtype)).astype(jnp.float32)
    maps, att = _find_call(feat, ohf, W_find, b_find.reshape(N_FIND, 1))

    ri = root_inst.astype(jnp.int32).reshape(B, 1)
    ys = yesno.astype(jnp.float32).reshape(B, 1)
    w2t = jnp.transpose(W_meas2, (0, 2, 1)).reshape(N_ROOT * N_ANS, HID)
    wdt = jnp.transpose(W_desc, (0, 2, 1)).reshape(N_ROOT * N_ANS, C)
    rlt = _root_call(maps, att, ri, ys, W_meas1,
                     b_meas1.reshape(N_ROOT, 1, HID), w2t, wdt)

    emb = _emb_gather(embed, question.reshape(-1).astype(jnp.int32))
    embt = emb.reshape(B, L, D).transpose(1, 0, 2).reshape(L * B, D)
    idx = (jnp.clip(length, 1, L) - 1).astype(jnp.int32).reshape(B, 1)
    return _lstm_call(embt, Wi, Wh, b_lstm.reshape(1, 4 * D), idx,
                      W_out, b_out.reshape(1, N_ANS), rlt, ri, ys,
                      b_meas2, b_desc)
